# baseline (speedup 1.0000x reference)
import functools

import numpy as np
import jax
import jax.numpy as jnp
from jax import lax
from jax.experimental import pallas as pl
from jax.experimental.pallas import tpu as pltpu
from jax.experimental.pallas import tpu_sc as plsc

_NUM_LEVELS = 16
_BASE_RES = 16
_MAX_PARAMS = 2 ** 19
_DESIRED_RES = 2048
_N = 1048576
_PER_LEVEL_SCALE = float(np.exp2(np.log2(_DESIRED_RES / _BASE_RES) / (_NUM_LEVELS - 1)))
_P1 = int(np.uint32(2654435761).astype(np.int32))
_P2 = int(np.uint32(805459861).astype(np.int32))


def _levels():
    scales, resolutions, offsets, sizes = [], [], [], []
    offset = 0
    for l in range(_NUM_LEVELS):
        scale = _BASE_RES * (_PER_LEVEL_SCALE ** l) - 1.0
        res = int(np.ceil(scale)) + 1
        params = min(_MAX_PARAMS, res ** 3)
        params = int(np.ceil(params / 8) * 8)
        scales.append(scale)
        resolutions.append(res)
        offsets.append(offset)
        sizes.append(params)
        offset += params
    return scales, resolutions, offsets, sizes


_SCALES, _RES, _OFF, _SIZES = _levels()
_USE_HASH = [(r ** 3) > s for r, s in zip(_RES, _SIZES)]

_NC, _NS = 2, 16
_NW = _NC * _NS
_C = 512
_NPW = _N // _NW
_NCHUNK = _NPW // _C


def _enc_body(x0h, x1h, x2h, tabh, feath, xv0, xv1, xv2,
              idxv0, idxv1, colv0, colv1, rowsv0, rowsv1, featv, sem0, sem1):
    wid = lax.axis_index("s") * _NC + lax.axis_index("c")
    sems = (sem0, sem1)
    idxvs = (idxv0, idxv1)
    colvs = (colv0, colv1)
    rowsvs = (rowsv0, rowsv1)
    iota = lax.iota(jnp.int32, 16)

    def idx_phase(l, b):
        scale = jnp.float32(_SCALES[l])
        off = _OFF[l]
        idxv = idxvs[b]
        colv = colvs[b]

        def jb(j, carry):
            o = pl.multiple_of(j * 16, 16)
            px = xv0[pl.ds(o, 16)] * scale + 0.5
            py = xv1[pl.ds(o, 16)] * scale + 0.5
            pz = xv2[pl.ds(o, 16)] * scale + 0.5
            gx = px.astype(jnp.int32)
            gy = py.astype(jnp.int32)
            gz = pz.astype(jnp.int32)
            gx1 = gx + 1
            gy1 = gy + 1
            gz1 = gz + 1
            if _USE_HASH[l]:
                m = _SIZES[l] - 1
                hy0 = gy * _P1
                hy1 = gy1 * _P1
                hz0 = gz * _P2
                hz1 = gz1 * _P2
                xy = (gx ^ hy0, gx1 ^ hy0, gx ^ hy1, gx1 ^ hy1)
                hz = (hz0, hz1)
                for c in range(8):
                    idx = ((xy[c & 3] ^ hz[c >> 2]) & m) + off
                    idxv[pl.ds(c * _C + o, 16)] = idx >> 2
                    colv[pl.ds(c * _C + o, 16)] = (idx & 3) << 1
            else:
                res = _RES[l]
                sy0 = gy * res
                sy1 = gy1 * res
                sz0 = gz * (res * res)
                sz1 = gz1 * (res * res)
                yz = (sy0 + sz0, sy1 + sz0, sy0 + sz1, sy1 + sz1)
                gxs = (gx, gx1)
                for c in range(8):
                    idx = gxs[c & 1] + yz[c >> 1] + off
                    idxv[pl.ds(c * _C + o, 16)] = idx >> 2
                    colv[pl.ds(c * _C + o, 16)] = (idx & 3) << 1
            return carry

        lax.fori_loop(0, _C // 16, jb, 0, unroll=2)

    def acc_phase(l, b):
        scale = jnp.float32(_SCALES[l])
        rows = rowsvs[b]
        colv = colvs[b]
        cl0 = jnp.full((16,), 2 * l, jnp.int32)
        cl1 = jnp.full((16,), 2 * l + 1, jnp.int32)

        def jb(j, carry):
            o = pl.multiple_of(j * 16, 16)
            px = xv0[pl.ds(o, 16)] * scale + 0.5
            py = xv1[pl.ds(o, 16)] * scale + 0.5
            pz = xv2[pl.ds(o, 16)] * scale + 0.5
            fx = px - px.astype(jnp.int32).astype(jnp.float32)
            fy = py - py.astype(jnp.int32).astype(jnp.float32)
            fz = pz - pz.astype(jnp.int32).astype(jnp.float32)
            wx = (1.0 - fx, fx)
            wy = (1.0 - fy, fy)
            wz = (1.0 - fz, fz)
            wxy = (wx[0] * wy[0], wx[1] * wy[0], wx[0] * wy[1], wx[1] * wy[1])
            acc0 = None
            acc1 = None
            for c in range(8):
                w = wxy[c & 3] * wz[c >> 2]
                ridx = iota + (c * _C + o)
                cbase = colv[pl.ds(c * _C + o, 16)]
                r0 = plsc.load_gather(rows, [ridx, cbase])
                r1 = plsc.load_gather(rows, [ridx, cbase + 1])
                if c == 0:
                    acc0 = w * r0
                    acc1 = w * r1
                else:
                    acc0 = acc0 + w * r0
                    acc1 = acc1 + w * r1
            prow = iota + o
            plsc.store_scatter(featv, [prow, cl0], acc0)
            plsc.store_scatter(featv, [prow, cl1], acc1)
            return carry

        lax.fori_loop(0, _C // 16, jb, 0, unroll=2)

    def chunk_body(ci, carry):
        base = pl.multiple_of(wid * _NPW + ci * _C, _C)
        pltpu.sync_copy(x0h.at[pl.ds(base, _C)], xv0)
        pltpu.sync_copy(x1h.at[pl.ds(base, _C)], xv1)
        pltpu.sync_copy(x2h.at[pl.ds(base, _C)], xv2)

        idx_phase(0, 0)
        handles = [None, None]
        handles[0] = pltpu.async_copy(tabh.at[idxvs[0]], rowsvs[0], sems[0])
        for l in range(1, _NUM_LEVELS):
            b = l % 2
            bp = (l - 1) % 2
            idx_phase(l, b)
            handles[b] = pltpu.async_copy(tabh.at[idxvs[b]], rowsvs[b], sems[b])
            handles[bp].wait()
            acc_phase(l - 1, bp)
        handles[(_NUM_LEVELS - 1) % 2].wait()
        acc_phase(_NUM_LEVELS - 1, (_NUM_LEVELS - 1) % 2)

        pltpu.sync_copy(featv, feath.at[pl.ds(base, _C)])
        return carry

    lax.fori_loop(0, _NCHUNK, chunk_body, 0)


@functools.partial(
    pl.kernel,
    out_type=jax.ShapeDtypeStruct((_N, 32), jnp.float32),
    mesh=plsc.VectorSubcoreMesh(core_axis_name="c", subcore_axis_name="s"),
    scratch_types=[
        pltpu.VMEM((_C,), jnp.float32),
        pltpu.VMEM((_C,), jnp.float32),
        pltpu.VMEM((_C,), jnp.float32),
        pltpu.VMEM((8 * _C,), jnp.int32),
        pltpu.VMEM((8 * _C,), jnp.int32),
        pltpu.VMEM((8 * _C,), jnp.int32),
        pltpu.VMEM((8 * _C,), jnp.int32),
        pltpu.VMEM((8 * _C, 8), jnp.float32),
        pltpu.VMEM((8 * _C, 8), jnp.float32),
        pltpu.VMEM((_C, 32), jnp.float32),
        pltpu.SemaphoreType.DMA,
        pltpu.SemaphoreType.DMA,
    ],
    compiler_params=pltpu.CompilerParams(
        needs_layout_passes=False, use_tc_tiling_on_sc=False),
)
def _encode(*args):
    _enc_body(*args)


_B = 4096


def _mlp_body(fref, w0r, w1r, w2r, w3r, w4r, b0r, b1r, b2r, b3r, b4r, oref):
    dn = (((1,), (1,)), ((), ()))
    h = fref[...]
    h = jnp.maximum(
        lax.dot_general(h, w0r[...], dn, preferred_element_type=jnp.float32)
        + b0r[...], 0.0)
    h = jnp.maximum(
        lax.dot_general(h, w1r[...], dn, preferred_element_type=jnp.float32)
        + b1r[...], 0.0)
    h = jnp.maximum(
        lax.dot_general(h, w2r[...], dn, preferred_element_type=jnp.float32)
        + b2r[...], 0.0)
    h = jnp.maximum(
        lax.dot_general(h, w3r[...], dn, preferred_element_type=jnp.float32)
        + b3r[...], 0.0)
    out8 = lax.dot_general(h, w4r[...], dn, preferred_element_type=jnp.float32)
    oref[...] = out8[:, 0:1] + b4r[0, 0]


def _full_spec(shape):
    nd = len(shape)
    return pl.BlockSpec(shape, lambda i: (0,) * nd)


def _mlp(feats, W0, W1, W2, W3, W4, b0, b1, b2, b3, b4):
    grid = (_N // _B,)
    return pl.pallas_call(
        _mlp_body,
        grid=grid,
        in_specs=[
            pl.BlockSpec((_B, 32), lambda i: (i, 0)),
            _full_spec(W0.shape), _full_spec(W1.shape), _full_spec(W2.shape),
            _full_spec(W3.shape), _full_spec(W4.shape),
            _full_spec(b0.shape), _full_spec(b1.shape), _full_spec(b2.shape),
            _full_spec(b3.shape),
            pl.BlockSpec(memory_space=pltpu.SMEM),
        ],
        out_specs=pl.BlockSpec((_B, 1), lambda i: (i, 0)),
        out_shape=jax.ShapeDtypeStruct((_N, 1), jnp.float32),
    )(feats, W0, W1, W2, W3, W4, b0, b1, b2, b3, b4)


def kernel(x, table, W0, b0, W1, b1, W2, b2, W3, b3, W4, b4):
    x0 = x[:, 0]
    x1 = x[:, 1]
    x2 = x[:, 2]
    tab4 = table.reshape(-1, 8)
    feats = _encode(x0, x1, x2, tab4)
    W4p = jnp.pad(W4, ((0, 7), (0, 0)))
    return _mlp(
        feats, W0, W1, W2, W3, W4p,
        b0.reshape(1, -1), b1.reshape(1, -1), b2.reshape(1, -1),
        b3.reshape(1, -1), b4.reshape(1, 1),
    )


# speedup vs baseline: 1.2074x; 1.2074x over previous
import functools

import numpy as np
import jax
import jax.numpy as jnp
from jax import lax
from jax.experimental import pallas as pl
from jax.experimental.pallas import tpu as pltpu
from jax.experimental.pallas import tpu_sc as plsc

_NUM_LEVELS = 16
_BASE_RES = 16
_MAX_PARAMS = 2 ** 19
_DESIRED_RES = 2048
_N = 1048576
_PER_LEVEL_SCALE = float(np.exp2(np.log2(_DESIRED_RES / _BASE_RES) / (_NUM_LEVELS - 1)))
_P1 = int(np.uint32(2654435761).astype(np.int32))
_P2 = int(np.uint32(805459861).astype(np.int32))


def _levels():
    scales, resolutions, offsets, sizes = [], [], [], []
    offset = 0
    for l in range(_NUM_LEVELS):
        scale = _BASE_RES * (_PER_LEVEL_SCALE ** l) - 1.0
        res = int(np.ceil(scale)) + 1
        params = min(_MAX_PARAMS, res ** 3)
        params = int(np.ceil(params / 8) * 8)
        scales.append(scale)
        resolutions.append(res)
        offsets.append(offset)
        sizes.append(params)
        offset += params
    return scales, resolutions, offsets, sizes


_SCALES, _RES, _OFF, _SIZES = _levels()
_USE_HASH = [(r ** 3) > s for r, s in zip(_RES, _SIZES)]

_NC, _NS = 2, 16
_NW = _NC * _NS
_C = 1024
_NPW = _N // _NW
_NCHUNK = _NPW // _C


def _enc_body(x0h, x1h, x2h, t0h, t1h, feath, xv0, xv1, xv2,
              idxv0, idxv1, r0a, r1a, r0b, r1b, featv,
              semA0, semA1, semB0, semB1):
    wid = lax.axis_index("s") * _NC + lax.axis_index("c")
    idxvs = (idxv0, idxv1)
    rows = ((r0a, r1a), (r0b, r1b))
    sems = ((semA0, semA1), (semB0, semB1))
    iota = lax.iota(jnp.int32, 16)

    def idx_phase(l, b):
        scale = jnp.float32(_SCALES[l])
        off = _OFF[l]
        idxv = idxvs[b]

        def jb(j, carry):
            o = pl.multiple_of(j * 16, 16)
            px = xv0[pl.ds(o, 16)] * scale + 0.5
            py = xv1[pl.ds(o, 16)] * scale + 0.5
            pz = xv2[pl.ds(o, 16)] * scale + 0.5
            gx = px.astype(jnp.int32)
            gy = py.astype(jnp.int32)
            gz = pz.astype(jnp.int32)
            gx1 = gx + 1
            gy1 = gy + 1
            gz1 = gz + 1
            if _USE_HASH[l]:
                m = _SIZES[l] - 1
                hy0 = gy * _P1
                hy1 = gy1 * _P1
                hz0 = gz * _P2
                hz1 = gz1 * _P2
                xy = (gx ^ hy0, gx1 ^ hy0, gx ^ hy1, gx1 ^ hy1)
                hz = (hz0, hz1)
                for c in range(8):
                    idx = ((xy[c & 3] ^ hz[c >> 2]) & m) + off
                    idxv[pl.ds(c * _C + o, 16)] = idx
            else:
                res = _RES[l]
                sy0 = gy * res
                sy1 = gy1 * res
                sz0 = gz * (res * res)
                sz1 = gz1 * (res * res)
                yz = (sy0 + sz0, sy1 + sz0, sy0 + sz1, sy1 + sz1)
                gxs = (gx, gx1)
                for c in range(8):
                    idx = gxs[c & 1] + yz[c >> 1] + off
                    idxv[pl.ds(c * _C + o, 16)] = idx
            return carry

        lax.fori_loop(0, _C // 16, jb, 0, unroll=2)

    def fire(b):
        idxv = idxvs[b]
        h0 = pltpu.async_copy(t0h.at[idxv], rows[b][0], sems[b][0])
        h1 = pltpu.async_copy(t1h.at[idxv], rows[b][1], sems[b][1])
        return (h0, h1)

    def acc_phase(l, b):
        scale = jnp.float32(_SCALES[l])
        rows0, rows1 = rows[b]
        cl0 = jnp.full((16,), 2 * l, jnp.int32)
        cl1 = jnp.full((16,), 2 * l + 1, jnp.int32)

        def jb(j, carry):
            o = pl.multiple_of(j * 16, 16)
            px = xv0[pl.ds(o, 16)] * scale + 0.5
            py = xv1[pl.ds(o, 16)] * scale + 0.5
            pz = xv2[pl.ds(o, 16)] * scale + 0.5
            fx = px - px.astype(jnp.int32).astype(jnp.float32)
            fy = py - py.astype(jnp.int32).astype(jnp.float32)
            fz = pz - pz.astype(jnp.int32).astype(jnp.float32)
            wx = (1.0 - fx, fx)
            wy = (1.0 - fy, fy)
            wz = (1.0 - fz, fz)
            wxy = (wx[0] * wy[0], wx[1] * wy[0], wx[0] * wy[1], wx[1] * wy[1])
            acc0 = None
            acc1 = None
            for c in range(8):
                w = wxy[c & 3] * wz[c >> 2]
                r0 = rows0[pl.ds(c * _C + o, 16)]
                r1 = rows1[pl.ds(c * _C + o, 16)]
                if c == 0:
                    acc0 = w * r0
                    acc1 = w * r1
                else:
                    acc0 = acc0 + w * r0
                    acc1 = acc1 + w * r1
            prow = iota + o
            plsc.store_scatter(featv, [prow, cl0], acc0)
            plsc.store_scatter(featv, [prow, cl1], acc1)
            return carry

        lax.fori_loop(0, _C // 16, jb, 0, unroll=2)

    def chunk_body(ci, carry):
        base = pl.multiple_of(wid * _NPW + ci * _C, _C)
        pltpu.sync_copy(x0h.at[pl.ds(base, _C)], xv0)
        pltpu.sync_copy(x1h.at[pl.ds(base, _C)], xv1)
        pltpu.sync_copy(x2h.at[pl.ds(base, _C)], xv2)

        idx_phase(0, 0)
        handles = [None, None]
        handles[0] = fire(0)
        for l in range(1, _NUM_LEVELS):
            b = l % 2
            bp = (l - 1) % 2
            idx_phase(l, b)
            handles[b] = fire(b)
            handles[bp][0].wait()
            handles[bp][1].wait()
            acc_phase(l - 1, bp)
        bl = (_NUM_LEVELS - 1) % 2
        handles[bl][0].wait()
        handles[bl][1].wait()
        acc_phase(_NUM_LEVELS - 1, bl)

        pltpu.sync_copy(featv, feath.at[pl.ds(base, _C)])
        return carry

    lax.fori_loop(0, _NCHUNK, chunk_body, 0)


@functools.partial(
    pl.kernel,
    out_type=jax.ShapeDtypeStruct((_N, 32), jnp.float32),
    mesh=plsc.VectorSubcoreMesh(core_axis_name="c", subcore_axis_name="s"),
    scratch_types=[
        pltpu.VMEM((_C,), jnp.float32),
        pltpu.VMEM((_C,), jnp.float32),
        pltpu.VMEM((_C,), jnp.float32),
        pltpu.VMEM((8 * _C,), jnp.int32),
        pltpu.VMEM((8 * _C,), jnp.int32),
        pltpu.VMEM((8 * _C,), jnp.float32),
        pltpu.VMEM((8 * _C,), jnp.float32),
        pltpu.VMEM((8 * _C,), jnp.float32),
        pltpu.VMEM((8 * _C,), jnp.float32),
        pltpu.VMEM((_C, 32), jnp.float32),
        pltpu.SemaphoreType.DMA,
        pltpu.SemaphoreType.DMA,
        pltpu.SemaphoreType.DMA,
        pltpu.SemaphoreType.DMA,
    ],
    compiler_params=pltpu.CompilerParams(
        needs_layout_passes=False, use_tc_tiling_on_sc=False),
)
def _encode(*args):
    _enc_body(*args)


_B = 4096


def _mlp_body(fref, w0r, w1r, w2r, w3r, w4r, b0r, b1r, b2r, b3r, b4r, oref):
    dn = (((1,), (1,)), ((), ()))
    h = fref[...]
    h = jnp.maximum(
        lax.dot_general(h, w0r[...], dn, preferred_element_type=jnp.float32)
        + b0r[...], 0.0)
    h = jnp.maximum(
        lax.dot_general(h, w1r[...], dn, preferred_element_type=jnp.float32)
        + b1r[...], 0.0)
    h = jnp.maximum(
        lax.dot_general(h, w2r[...], dn, preferred_element_type=jnp.float32)
        + b2r[...], 0.0)
    h = jnp.maximum(
        lax.dot_general(h, w3r[...], dn, preferred_element_type=jnp.float32)
        + b3r[...], 0.0)
    out8 = lax.dot_general(h, w4r[...], dn, preferred_element_type=jnp.float32)
    oref[...] = out8[:, 0:1] + b4r[0, 0]


def _full_spec(shape):
    nd = len(shape)
    return pl.BlockSpec(shape, lambda i: (0,) * nd)


def _mlp(feats, W0, W1, W2, W3, W4, b0, b1, b2, b3, b4):
    grid = (_N // _B,)
    return pl.pallas_call(
        _mlp_body,
        grid=grid,
        in_specs=[
            pl.BlockSpec((_B, 32), lambda i: (i, 0)),
            _full_spec(W0.shape), _full_spec(W1.shape), _full_spec(W2.shape),
            _full_spec(W3.shape), _full_spec(W4.shape),
            _full_spec(b0.shape), _full_spec(b1.shape), _full_spec(b2.shape),
            _full_spec(b3.shape),
            pl.BlockSpec(memory_space=pltpu.SMEM),
        ],
        out_specs=pl.BlockSpec((_B, 1), lambda i: (i, 0)),
        out_shape=jax.ShapeDtypeStruct((_N, 1), jnp.float32),
    )(feats, W0, W1, W2, W3, W4, b0, b1, b2, b3, b4)


def kernel(x, table, W0, b0, W1, b1, W2, b2, W3, b3, W4, b4):
    x0 = x[:, 0]
    x1 = x[:, 1]
    x2 = x[:, 2]
    t0 = table[:, 0]
    t1 = table[:, 1]
    feats = _encode(x0, x1, x2, t0, t1)
    W4p = jnp.pad(W4, ((0, 7), (0, 0)))
    return _mlp(
        feats, W0, W1, W2, W3, W4p,
        b0.reshape(1, -1), b1.reshape(1, -1), b2.reshape(1, -1),
        b3.reshape(1, -1), b4.reshape(1, 1),
    )


# speedup vs baseline: 2.0167x; 1.6702x over previous
import functools

import numpy as np
import jax
import jax.numpy as jnp
from jax import lax
from jax.experimental import pallas as pl
from jax.experimental.pallas import tpu as pltpu
from jax.experimental.pallas import tpu_sc as plsc

_NUM_LEVELS = 16
_BASE_RES = 16
_MAX_PARAMS = 2 ** 19
_DESIRED_RES = 2048
_N = 1048576
_PER_LEVEL_SCALE = float(np.exp2(np.log2(_DESIRED_RES / _BASE_RES) / (_NUM_LEVELS - 1)))
_P1 = int(np.uint32(2654435761).astype(np.int32))
_P2 = int(np.uint32(805459861).astype(np.int32))


def _levels():
    scales, resolutions, offsets, sizes = [], [], [], []
    offset = 0
    for l in range(_NUM_LEVELS):
        scale = _BASE_RES * (_PER_LEVEL_SCALE ** l) - 1.0
        res = int(np.ceil(scale)) + 1
        params = min(_MAX_PARAMS, res ** 3)
        params = int(np.ceil(params / 8) * 8)
        scales.append(scale)
        resolutions.append(res)
        offsets.append(offset)
        sizes.append(params)
        offset += params
    return scales, resolutions, offsets, sizes


_SCALES, _RES, _OFF, _SIZES = _levels()
_USE_HASH = [(r ** 3) > s for r, s in zip(_RES, _SIZES)]
_T = _OFF[-1] + _SIZES[-1]

_NC, _NS = 2, 16
_NW = _NC * _NS
_C = 256
_NPW = _N // _NW
_NCHUNK = _NPW // _C

_IC = 512
_TP = ((_T + _NS * _IC - 1) // (_NS * _IC)) * (_NS * _IC)
_IPW = _TP // _NS
_ISTEPS = _IPW // _IC
_NLOC = _OFF[2]


def _enc_body(x0h, x1h, x2h, t0h, t1h, feath, tabi,
              xv0, xv1, xv2, idxv0, idxv1, colv0, colv1, ra, rb, featv,
              t01a, t01b, tv0, tv1, ibuf, semA, semB):
    cid = lax.axis_index("c")
    sid = lax.axis_index("s")
    wid = sid * _NC + cid
    idxvs = (idxv0, idxv1)
    colvs = (colv0, colv1)
    rows = (ra, rb)
    sems = (semA, semB)
    iota = lax.iota(jnp.int32, 16)
    col0 = jnp.zeros((16,), jnp.int32)
    col1 = jnp.ones((16,), jnp.int32)
    tbase4 = cid * (_TP // 4)

    pltpu.sync_copy(t0h.at[pl.ds(0, _NLOC)], t01a)
    pltpu.sync_copy(t1h.at[pl.ds(0, _NLOC)], t01b)

    def istep(k, carry):
        start = pl.multiple_of(sid * _IPW + k * _IC, _IC)
        pltpu.sync_copy(t0h.at[pl.ds(start, _IC)], tv0)
        pltpu.sync_copy(t1h.at[pl.ds(start, _IC)], tv1)

        def jb(j, c2):
            o = pl.multiple_of(j * 16, 16)
            rr = iota + o
            rq = rr >> 2
            rk = (rr & 3) * 2
            plsc.store_scatter(ibuf, [rq, rk], tv0[pl.ds(o, 16)])
            plsc.store_scatter(ibuf, [rq, rk + 1], tv1[pl.ds(o, 16)])
            return c2

        lax.fori_loop(0, _IC // 16, jb, 0, unroll=4)
        pltpu.sync_copy(
            ibuf, tabi.at[pl.ds(tbase4 + (sid * _IPW + k * _IC) // 4, _IC // 4)])
        return carry

    lax.fori_loop(0, _ISTEPS, istep, 0)
    plsc.subcore_barrier()

    def point_vregs(l, o):
        scale = jnp.float32(_SCALES[l])
        px = xv0[pl.ds(o, 16)] * scale + 0.5
        py = xv1[pl.ds(o, 16)] * scale + 0.5
        pz = xv2[pl.ds(o, 16)] * scale + 0.5
        gx = px.astype(jnp.int32)
        gy = py.astype(jnp.int32)
        gz = pz.astype(jnp.int32)
        return px, py, pz, gx, gy, gz

    def corner_indices(l, gx, gy, gz, off):
        gx1 = gx + 1
        gy1 = gy + 1
        gz1 = gz + 1
        if _USE_HASH[l]:
            m = _SIZES[l] - 1
            hy0 = gy * _P1
            hy1 = gy1 * _P1
            hz0 = gz * _P2
            hz1 = gz1 * _P2
            xy = (gx ^ hy0, gx1 ^ hy0, gx ^ hy1, gx1 ^ hy1)
            hz = (hz0, hz1)
            return [((xy[c & 3] ^ hz[c >> 2]) & m) + off for c in range(8)]
        res = _RES[l]
        sy0 = gy * res
        sy1 = gy1 * res
        sz0 = gz * (res * res)
        sz1 = gz1 * (res * res)
        yz = (sy0 + sz0, sy1 + sz0, sy0 + sz1, sy1 + sz1)
        gxs = (gx, gx1)
        return [gxs[c & 1] + yz[c >> 1] + off for c in range(8)]

    def weights(px, py, pz, gx, gy, gz):
        fx = px - gx.astype(jnp.float32)
        fy = py - gy.astype(jnp.float32)
        fz = pz - gz.astype(jnp.float32)
        wx = (1.0 - fx, fx)
        wy = (1.0 - fy, fy)
        wz = (1.0 - fz, fz)
        wxy = (wx[0] * wy[0], wx[1] * wy[0], wx[0] * wy[1], wx[1] * wy[1])
        return [wxy[c & 3] * wz[c >> 2] for c in range(8)]

    def scatter_out(l, o, acc0, acc1):
        prow = iota + o
        plsc.store_scatter(featv, [prow, jnp.full((16,), 2 * l, jnp.int32)], acc0)
        plsc.store_scatter(featv, [prow, jnp.full((16,), 2 * l + 1, jnp.int32)], acc1)

    def idx_phase(l, b):
        idxv = idxvs[b]
        colv = colvs[b]
        off = _OFF[l]

        def jb(j, carry):
            o = pl.multiple_of(j * 16, 16)
            _, _, _, gx, gy, gz = point_vregs(l, o)
            idxs = corner_indices(l, gx, gy, gz, off)
            for c in range(8):
                idxv[pl.ds(c * _C + o, 16)] = (idxs[c] >> 2) + tbase4
                colv[pl.ds(c * _C + o, 16)] = (idxs[c] & 3) * 2
            return carry

        lax.fori_loop(0, _C // 16, jb, 0, unroll=2)

    def fire(b):
        return pltpu.async_copy(tabi.at[idxvs[b]], rows[b], sems[b])

    def acc_phase(l, b):
        rv = rows[b]
        colv = colvs[b]

        def jb(j, carry):
            o = pl.multiple_of(j * 16, 16)
            px, py, pz, gx, gy, gz = point_vregs(l, o)
            ws = weights(px, py, pz, gx, gy, gz)
            acc0 = None
            acc1 = None
            for c in range(8):
                ridx = iota + (c * _C + o)
                cb = colv[pl.ds(c * _C + o, 16)]
                r0 = plsc.load_gather(rv, [ridx, cb])
                r1 = plsc.load_gather(rv, [ridx, cb + 1])
                if c == 0:
                    acc0 = ws[c] * r0
                    acc1 = ws[c] * r1
                else:
                    acc0 = acc0 + ws[c] * r0
                    acc1 = acc1 + ws[c] * r1
            scatter_out(l, o, acc0, acc1)
            return carry

        lax.fori_loop(0, _C // 16, jb, 0, unroll=2)

    def acc_local(l):
        off = _OFF[l]

        def jb(j, carry):
            o = pl.multiple_of(j * 16, 16)
            px, py, pz, gx, gy, gz = point_vregs(l, o)
            idxs = corner_indices(l, gx, gy, gz, off)
            ws = weights(px, py, pz, gx, gy, gz)
            acc0 = None
            acc1 = None
            for c in range(8):
                r0 = plsc.load_gather(t01a, [idxs[c]])
                r1 = plsc.load_gather(t01b, [idxs[c]])
                if c == 0:
                    acc0 = ws[c] * r0
                    acc1 = ws[c] * r1
                else:
                    acc0 = acc0 + ws[c] * r0
                    acc1 = acc1 + ws[c] * r1
            scatter_out(l, o, acc0, acc1)
            return carry

        lax.fori_loop(0, _C // 16, jb, 0, unroll=2)

    def chunk_body(ci, carry):
        base = pl.multiple_of(wid * _NPW + ci * _C, _C)
        pltpu.sync_copy(x0h.at[pl.ds(base, _C)], xv0)
        pltpu.sync_copy(x1h.at[pl.ds(base, _C)], xv1)
        pltpu.sync_copy(x2h.at[pl.ds(base, _C)], xv2)

        idx_phase(2, 0)
        handles = [None, None]
        handles[0] = fire(0)
        acc_local(0)
        acc_local(1)
        for l in range(3, _NUM_LEVELS):
            b = (l - 2) % 2
            bp = (l - 3) % 2
            idx_phase(l, b)
            handles[b] = fire(b)
            handles[bp].wait()
            acc_phase(l - 1, bp)
        bl = (_NUM_LEVELS - 3) % 2
        handles[bl].wait()
        acc_phase(_NUM_LEVELS - 1, bl)

        pltpu.sync_copy(featv, feath.at[pl.ds(base, _C)])
        return carry

    lax.fori_loop(0, _NCHUNK, chunk_body, 0)


@functools.partial(
    pl.kernel,
    out_type=jax.ShapeDtypeStruct((_N, 32), jnp.float32),
    mesh=plsc.VectorSubcoreMesh(core_axis_name="c", subcore_axis_name="s"),
    scratch_types=[
        pltpu.HBM((_NC * _TP // 4, 8), jnp.float32),
        pltpu.VMEM((_C,), jnp.float32),
        pltpu.VMEM((_C,), jnp.float32),
        pltpu.VMEM((_C,), jnp.float32),
        pltpu.VMEM((8 * _C,), jnp.int32),
        pltpu.VMEM((8 * _C,), jnp.int32),
        pltpu.VMEM((8 * _C,), jnp.int32),
        pltpu.VMEM((8 * _C,), jnp.int32),
        pltpu.VMEM((8 * _C, 8), jnp.float32),
        pltpu.VMEM((8 * _C, 8), jnp.float32),
        pltpu.VMEM((_C, 32), jnp.float32),
        pltpu.VMEM((_NLOC,), jnp.float32),
        pltpu.VMEM((_NLOC,), jnp.float32),
        pltpu.VMEM((_IC,), jnp.float32),
        pltpu.VMEM((_IC,), jnp.float32),
        pltpu.VMEM((_IC // 4, 8), jnp.float32),
        pltpu.SemaphoreType.DMA,
        pltpu.SemaphoreType.DMA,
    ],
    compiler_params=pltpu.CompilerParams(
        needs_layout_passes=False, use_tc_tiling_on_sc=False),
)
def _encode(*args):
    _enc_body(*args)


_B = 4096


def _mlp_body(fref, w0r, w1r, w2r, w3r, w4r, b0r, b1r, b2r, b3r, b4r, oref):
    dn = (((1,), (1,)), ((), ()))
    h = fref[...]
    h = jnp.maximum(
        lax.dot_general(h, w0r[...], dn, preferred_element_type=jnp.float32)
        + b0r[...], 0.0)
    h = jnp.maximum(
        lax.dot_general(h, w1r[...], dn, preferred_element_type=jnp.float32)
        + b1r[...], 0.0)
    h = jnp.maximum(
        lax.dot_general(h, w2r[...], dn, preferred_element_type=jnp.float32)
        + b2r[...], 0.0)
    h = jnp.maximum(
        lax.dot_general(h, w3r[...], dn, preferred_element_type=jnp.float32)
        + b3r[...], 0.0)
    out8 = lax.dot_general(h, w4r[...], dn, preferred_element_type=jnp.float32)
    oref[...] = out8[:, 0:1] + b4r[0, 0]


def _full_spec(shape):
    nd = len(shape)
    return pl.BlockSpec(shape, lambda i: (0,) * nd)


def _mlp(feats, W0, W1, W2, W3, W4, b0, b1, b2, b3, b4):
    grid = (_N // _B,)
    return pl.pallas_call(
        _mlp_body,
        grid=grid,
        in_specs=[
            pl.BlockSpec((_B, 32), lambda i: (i, 0)),
            _full_spec(W0.shape), _full_spec(W1.shape), _full_spec(W2.shape),
            _full_spec(W3.shape), _full_spec(W4.shape),
            _full_spec(b0.shape), _full_spec(b1.shape), _full_spec(b2.shape),
            _full_spec(b3.shape),
            pl.BlockSpec(memory_space=pltpu.SMEM),
        ],
        out_specs=pl.BlockSpec((_B, 1), lambda i: (i, 0)),
        out_shape=jax.ShapeDtypeStruct((_N, 1), jnp.float32),
    )(feats, W0, W1, W2, W3, W4, b0, b1, b2, b3, b4)


def kernel(x, table, W0, b0, W1, b1, W2, b2, W3, b3, W4, b4):
    x0 = x[:, 0]
    x1 = x[:, 1]
    x2 = x[:, 2]
    pad = _TP - _T
    t0 = jnp.pad(table[:, 0], (0, pad))
    t1 = jnp.pad(table[:, 1], (0, pad))
    feats = _encode(x0, x1, x2, t0, t1)
    W4p = jnp.pad(W4, ((0, 7), (0, 0)))
    return _mlp(
        feats, W0, W1, W2, W3, W4p,
        b0.reshape(1, -1), b1.reshape(1, -1), b2.reshape(1, -1),
        b3.reshape(1, -1), b4.reshape(1, 1),
    )


# speedup vs baseline: 2.3041x; 1.1425x over previous
import functools

import numpy as np
import jax
import jax.numpy as jnp
from jax import lax
from jax.experimental import pallas as pl
from jax.experimental.pallas import tpu as pltpu
from jax.experimental.pallas import tpu_sc as plsc

_NUM_LEVELS = 16
_BASE_RES = 16
_MAX_PARAMS = 2 ** 19
_DESIRED_RES = 2048
_N = 1048576
_PER_LEVEL_SCALE = float(np.exp2(np.log2(_DESIRED_RES / _BASE_RES) / (_NUM_LEVELS - 1)))
_P1 = int(np.uint32(2654435761).astype(np.int32))
_P2 = int(np.uint32(805459861).astype(np.int32))


def _levels():
    scales, resolutions, offsets, sizes = [], [], [], []
    offset = 0
    for l in range(_NUM_LEVELS):
        scale = _BASE_RES * (_PER_LEVEL_SCALE ** l) - 1.0
        res = int(np.ceil(scale)) + 1
        params = min(_MAX_PARAMS, res ** 3)
        params = int(np.ceil(params / 8) * 8)
        scales.append(scale)
        resolutions.append(res)
        offsets.append(offset)
        sizes.append(params)
        offset += params
    return scales, resolutions, offsets, sizes


_SCALES, _RES, _OFF, _SIZES = _levels()
_USE_HASH = [(r ** 3) > s for r, s in zip(_RES, _SIZES)]
_T = _OFF[-1] + _SIZES[-1]

_NC, _NS = 2, 16
_NW = _NC * _NS
_C = 256
_NSPLIT = 2
_NH = _N // _NSPLIT
_NPW = _NH // _NW
_NCHUNK = _NPW // _C

_IC = 512
_TP = ((_T + _NW * _IC - 1) // (_NW * _IC)) * (_NW * _IC)
_IPW = _TP // _NW
_ISTEPS = _IPW // _IC
_NLOC = _OFF[2]


def _enc_body(x0h, x1h, x2h, t0h, t1h, tabi, feath,
              xv0, xv1, xv2, idxv0, idxv1, colv0, colv1, ra, rb, featv,
              t01a, t01b, semA, semB):
    cid = lax.axis_index("c")
    sid = lax.axis_index("s")
    wid = sid * _NC + cid
    idxvs = (idxv0, idxv1)
    colvs = (colv0, colv1)
    rows = (ra, rb)
    sems = (semA, semB)
    iota = lax.iota(jnp.int32, 16)
    col0 = jnp.zeros((16,), jnp.int32)
    col1 = jnp.ones((16,), jnp.int32)
    tbase4 = 0

    pltpu.sync_copy(t0h.at[pl.ds(0, _NLOC)], t01a)
    pltpu.sync_copy(t1h.at[pl.ds(0, _NLOC)], t01b)

    def point_vregs(l, o):
        scale = jnp.float32(_SCALES[l])
        px = xv0[pl.ds(o, 16)] * scale + 0.5
        py = xv1[pl.ds(o, 16)] * scale + 0.5
        pz = xv2[pl.ds(o, 16)] * scale + 0.5
        gx = px.astype(jnp.int32)
        gy = py.astype(jnp.int32)
        gz = pz.astype(jnp.int32)
        return px, py, pz, gx, gy, gz

    def corner_indices(l, gx, gy, gz, off):
        gx1 = gx + 1
        gy1 = gy + 1
        gz1 = gz + 1
        if _USE_HASH[l]:
            m = _SIZES[l] - 1
            hy0 = gy * _P1
            hy1 = gy1 * _P1
            hz0 = gz * _P2
            hz1 = gz1 * _P2
            xy = (gx ^ hy0, gx1 ^ hy0, gx ^ hy1, gx1 ^ hy1)
            hz = (hz0, hz1)
            return [((xy[c & 3] ^ hz[c >> 2]) & m) + off for c in range(8)]
        res = _RES[l]
        sy0 = gy * res
        sy1 = gy1 * res
        sz0 = gz * (res * res)
        sz1 = gz1 * (res * res)
        yz = (sy0 + sz0, sy1 + sz0, sy0 + sz1, sy1 + sz1)
        gxs = (gx, gx1)
        return [gxs[c & 1] + yz[c >> 1] + off for c in range(8)]

    def weights(px, py, pz, gx, gy, gz):
        fx = px - gx.astype(jnp.float32)
        fy = py - gy.astype(jnp.float32)
        fz = pz - gz.astype(jnp.float32)
        wx = (1.0 - fx, fx)
        wy = (1.0 - fy, fy)
        wz = (1.0 - fz, fz)
        wxy = (wx[0] * wy[0], wx[1] * wy[0], wx[0] * wy[1], wx[1] * wy[1])
        return [wxy[c & 3] * wz[c >> 2] for c in range(8)]

    def scatter_out(l, o, acc0, acc1):
        prow = iota + o
        plsc.store_scatter(featv, [prow, jnp.full((16,), 2 * l, jnp.int32)], acc0)
        plsc.store_scatter(featv, [prow, jnp.full((16,), 2 * l + 1, jnp.int32)], acc1)

    def idx_phase(l, b):
        idxv = idxvs[b]
        colv = colvs[b]
        off = _OFF[l]

        def jb(j, carry):
            o = pl.multiple_of(j * 16, 16)
            _, _, _, gx, gy, gz = point_vregs(l, o)
            idxs = corner_indices(l, gx, gy, gz, off)
            for c in range(8):
                idxv[pl.ds(c * _C + o, 16)] = (idxs[c] >> 2) + tbase4
                colv[pl.ds(c * _C + o, 16)] = (idxs[c] & 3) * 2
            return carry

        lax.fori_loop(0, _C // 16, jb, 0, unroll=2)

    def fire(b):
        return pltpu.async_copy(tabi.at[idxvs[b]], rows[b], sems[b])

    def acc_phase(l, b):
        rv = rows[b]
        colv = colvs[b]

        def jb(j, carry):
            o = pl.multiple_of(j * 16, 16)
            px, py, pz, gx, gy, gz = point_vregs(l, o)
            ws = weights(px, py, pz, gx, gy, gz)
            acc0 = None
            acc1 = None
            for c in range(8):
                ridx = iota + (c * _C + o)
                cb = colv[pl.ds(c * _C + o, 16)]
                r0 = plsc.load_gather(rv, [ridx, cb])
                r1 = plsc.load_gather(rv, [ridx, cb + 1])
                if c == 0:
                    acc0 = ws[c] * r0
                    acc1 = ws[c] * r1
                else:
                    acc0 = acc0 + ws[c] * r0
                    acc1 = acc1 + ws[c] * r1
            scatter_out(l, o, acc0, acc1)
            return carry

        lax.fori_loop(0, _C // 16, jb, 0, unroll=2)

    def acc_local(l):
        off = _OFF[l]

        def jb(j, carry):
            o = pl.multiple_of(j * 16, 16)
            px, py, pz, gx, gy, gz = point_vregs(l, o)
            idxs = corner_indices(l, gx, gy, gz, off)
            ws = weights(px, py, pz, gx, gy, gz)
            acc0 = None
            acc1 = None
            for c in range(8):
                r0 = plsc.load_gather(t01a, [idxs[c]])
                r1 = plsc.load_gather(t01b, [idxs[c]])
                if c == 0:
                    acc0 = ws[c] * r0
                    acc1 = ws[c] * r1
                else:
                    acc0 = acc0 + ws[c] * r0
                    acc1 = acc1 + ws[c] * r1
            scatter_out(l, o, acc0, acc1)
            return carry

        lax.fori_loop(0, _C // 16, jb, 0, unroll=2)

    def chunk_body(ci, carry):
        base = pl.multiple_of(wid * _NPW + ci * _C, _C)
        pltpu.sync_copy(x0h.at[pl.ds(base, _C)], xv0)
        pltpu.sync_copy(x1h.at[pl.ds(base, _C)], xv1)
        pltpu.sync_copy(x2h.at[pl.ds(base, _C)], xv2)

        idx_phase(2, 0)
        handles = [None, None]
        handles[0] = fire(0)
        acc_local(0)
        acc_local(1)
        for l in range(3, _NUM_LEVELS):
            b = (l - 2) % 2
            bp = (l - 3) % 2
            idx_phase(l, b)
            handles[b] = fire(b)
            handles[bp].wait()
            acc_phase(l - 1, bp)
        bl = (_NUM_LEVELS - 3) % 2
        handles[bl].wait()
        acc_phase(_NUM_LEVELS - 1, bl)

        pltpu.sync_copy(featv, feath.at[pl.ds(base, _C)])
        return carry

    lax.fori_loop(0, _NCHUNK, chunk_body, 0)


def _build_body(t0h, t1h, tabi, tv0, tv1, ibuf):
    cid = lax.axis_index("c")
    sid = lax.axis_index("s")
    wid = sid * _NC + cid
    iota = lax.iota(jnp.int32, 16)

    def istep(k, carry):
        start = pl.multiple_of(wid * _IPW + k * _IC, _IC)
        pltpu.sync_copy(t0h.at[pl.ds(start, _IC)], tv0)
        pltpu.sync_copy(t1h.at[pl.ds(start, _IC)], tv1)

        def jb(j, c2):
            o = pl.multiple_of(j * 16, 16)
            rr = iota + o
            rq = rr >> 2
            rk = (rr & 3) * 2
            plsc.store_scatter(ibuf, [rq, rk], tv0[pl.ds(o, 16)])
            plsc.store_scatter(ibuf, [rq, rk + 1], tv1[pl.ds(o, 16)])
            return c2

        lax.fori_loop(0, _IC // 16, jb, 0, unroll=4)
        pltpu.sync_copy(
            ibuf, tabi.at[pl.ds((wid * _IPW + k * _IC) // 4, _IC // 4)])
        return carry

    lax.fori_loop(0, _ISTEPS, istep, 0)


@functools.partial(
    pl.kernel,
    out_type=jax.ShapeDtypeStruct((_TP // 4, 8), jnp.float32),
    mesh=plsc.VectorSubcoreMesh(core_axis_name="c", subcore_axis_name="s"),
    scratch_types=[
        pltpu.VMEM((_IC,), jnp.float32),
        pltpu.VMEM((_IC,), jnp.float32),
        pltpu.VMEM((_IC // 4, 8), jnp.float32),
    ],
    compiler_params=pltpu.CompilerParams(
        needs_layout_passes=False, use_tc_tiling_on_sc=False),
)
def _build_tab(*args):
    _build_body(*args)


@functools.partial(
    pl.kernel,
    out_type=jax.ShapeDtypeStruct((_NH, 32), jnp.float32),
    mesh=plsc.VectorSubcoreMesh(core_axis_name="c", subcore_axis_name="s"),
    scratch_types=[
        pltpu.VMEM((_C,), jnp.float32),
        pltpu.VMEM((_C,), jnp.float32),
        pltpu.VMEM((_C,), jnp.float32),
        pltpu.VMEM((8 * _C,), jnp.int32),
        pltpu.VMEM((8 * _C,), jnp.int32),
        pltpu.VMEM((8 * _C,), jnp.int32),
        pltpu.VMEM((8 * _C,), jnp.int32),
        pltpu.VMEM((8 * _C, 8), jnp.float32),
        pltpu.VMEM((8 * _C, 8), jnp.float32),
        pltpu.VMEM((_C, 32), jnp.float32),
        pltpu.VMEM((_NLOC,), jnp.float32),
        pltpu.VMEM((_NLOC,), jnp.float32),
        pltpu.SemaphoreType.DMA,
        pltpu.SemaphoreType.DMA,
    ],
    compiler_params=pltpu.CompilerParams(
        needs_layout_passes=False, use_tc_tiling_on_sc=False),
)
def _encode(*args):
    _enc_body(*args)


_B = 4096


def _mlp_body(fref, w0r, w1r, w2r, w3r, w4r, b0r, b1r, b2r, b3r, b4r, oref):
    dn = (((1,), (1,)), ((), ()))
    h = fref[...]
    h = jnp.maximum(
        lax.dot_general(h, w0r[...], dn, preferred_element_type=jnp.float32)
        + b0r[...], 0.0)
    h = jnp.maximum(
        lax.dot_general(h, w1r[...], dn, preferred_element_type=jnp.float32)
        + b1r[...], 0.0)
    h = jnp.maximum(
        lax.dot_general(h, w2r[...], dn, preferred_element_type=jnp.float32)
        + b2r[...], 0.0)
    h = jnp.maximum(
        lax.dot_general(h, w3r[...], dn, preferred_element_type=jnp.float32)
        + b3r[...], 0.0)
    out8 = lax.dot_general(h, w4r[...], dn, preferred_element_type=jnp.float32)
    oref[...] = out8[:, 0:1] + b4r[0, 0]


def _full_spec(shape):
    nd = len(shape)
    return pl.BlockSpec(shape, lambda i: (0,) * nd)


def _mlp(feats, W0, W1, W2, W3, W4, b0, b1, b2, b3, b4):
    grid = (_NH // _B,)
    return pl.pallas_call(
        _mlp_body,
        grid=grid,
        in_specs=[
            pl.BlockSpec((_B, 32), lambda i: (i, 0)),
            _full_spec(W0.shape), _full_spec(W1.shape), _full_spec(W2.shape),
            _full_spec(W3.shape), _full_spec(W4.shape),
            _full_spec(b0.shape), _full_spec(b1.shape), _full_spec(b2.shape),
            _full_spec(b3.shape),
            pl.BlockSpec(memory_space=pltpu.SMEM),
        ],
        out_specs=pl.BlockSpec((_B, 1), lambda i: (i, 0)),
        out_shape=jax.ShapeDtypeStruct((_NH, 1), jnp.float32),
    )(feats, W0, W1, W2, W3, W4, b0, b1, b2, b3, b4)


def kernel(x, table, W0, b0, W1, b1, W2, b2, W3, b3, W4, b4):
    x0 = x[:, 0]
    x1 = x[:, 1]
    x2 = x[:, 2]
    pad = _TP - _T
    t0 = jnp.pad(table[:, 0], (0, pad))
    t1 = jnp.pad(table[:, 1], (0, pad))
    tabi = _build_tab(t0, t1)
    W4p = jnp.pad(W4, ((0, 7), (0, 0)))
    bs = (b0.reshape(1, -1), b1.reshape(1, -1), b2.reshape(1, -1),
          b3.reshape(1, -1), b4.reshape(1, 1))
    outs = []
    for h in range(_NSPLIT):
        sl = slice(h * _NH, (h + 1) * _NH)
        feats = _encode(x0[sl], x1[sl], x2[sl], t0, t1, tabi)
        outs.append(_mlp(feats, W0, W1, W2, W3, W4p, *bs))
    return jnp.concatenate(outs, axis=0)


# speedup vs baseline: 2.3344x; 1.0131x over previous
import functools

import numpy as np
import jax
import jax.numpy as jnp
from jax import lax
from jax.experimental import pallas as pl
from jax.experimental.pallas import tpu as pltpu
from jax.experimental.pallas import tpu_sc as plsc

_NUM_LEVELS = 16
_BASE_RES = 16
_MAX_PARAMS = 2 ** 19
_DESIRED_RES = 2048
_N = 1048576
_PER_LEVEL_SCALE = float(np.exp2(np.log2(_DESIRED_RES / _BASE_RES) / (_NUM_LEVELS - 1)))
_P1 = int(np.uint32(2654435761).astype(np.int32))
_P2 = int(np.uint32(805459861).astype(np.int32))


def _levels():
    scales, resolutions, offsets, sizes = [], [], [], []
    offset = 0
    for l in range(_NUM_LEVELS):
        scale = _BASE_RES * (_PER_LEVEL_SCALE ** l) - 1.0
        res = int(np.ceil(scale)) + 1
        params = min(_MAX_PARAMS, res ** 3)
        params = int(np.ceil(params / 8) * 8)
        scales.append(scale)
        resolutions.append(res)
        offsets.append(offset)
        sizes.append(params)
        offset += params
    return scales, resolutions, offsets, sizes


_SCALES, _RES, _OFF, _SIZES = _levels()
_USE_HASH = [(r ** 3) > s for r, s in zip(_RES, _SIZES)]
_T = _OFF[-1] + _SIZES[-1]

_NC, _NS = 2, 16
_NW = _NC * _NS
_C = 256
_NSPLIT = 4
_NH = _N // _NSPLIT
_NPW = _NH // _NW
_NCHUNK = _NPW // _C

_IC = 512
_TP = ((_T + _NW * _IC - 1) // (_NW * _IC)) * (_NW * _IC)
_IPW = _TP // _NW
_ISTEPS = _IPW // _IC
_NLOC = _OFF[2]


def _enc_body(x0h, x1h, x2h, t0h, t1h, tabi, feath,
              xv0, xv1, xv2, idxv0, idxv1, colv0, colv1, ra, rb, featv,
              t01a, t01b, semA, semB):
    cid = lax.axis_index("c")
    sid = lax.axis_index("s")
    wid = sid * _NC + cid
    idxvs = (idxv0, idxv1)
    colvs = (colv0, colv1)
    rows = (ra, rb)
    sems = (semA, semB)
    iota = lax.iota(jnp.int32, 16)
    col0 = jnp.zeros((16,), jnp.int32)
    col1 = jnp.ones((16,), jnp.int32)
    tbase4 = 0

    pltpu.sync_copy(t0h.at[pl.ds(0, _NLOC)], t01a)
    pltpu.sync_copy(t1h.at[pl.ds(0, _NLOC)], t01b)

    def point_vregs(l, o):
        scale = jnp.float32(_SCALES[l])
        px = xv0[pl.ds(o, 16)] * scale + 0.5
        py = xv1[pl.ds(o, 16)] * scale + 0.5
        pz = xv2[pl.ds(o, 16)] * scale + 0.5
        gx = px.astype(jnp.int32)
        gy = py.astype(jnp.int32)
        gz = pz.astype(jnp.int32)
        return px, py, pz, gx, gy, gz

    def corner_indices(l, gx, gy, gz, off):
        gx1 = gx + 1
        gy1 = gy + 1
        gz1 = gz + 1
        if _USE_HASH[l]:
            m = _SIZES[l] - 1
            hy0 = gy * _P1
            hy1 = gy1 * _P1
            hz0 = gz * _P2
            hz1 = gz1 * _P2
            xy = (gx ^ hy0, gx1 ^ hy0, gx ^ hy1, gx1 ^ hy1)
            hz = (hz0, hz1)
            return [((xy[c & 3] ^ hz[c >> 2]) & m) + off for c in range(8)]
        res = _RES[l]
        sy0 = gy * res
        sy1 = gy1 * res
        sz0 = gz * (res * res)
        sz1 = gz1 * (res * res)
        yz = (sy0 + sz0, sy1 + sz0, sy0 + sz1, sy1 + sz1)
        gxs = (gx, gx1)
        return [gxs[c & 1] + yz[c >> 1] + off for c in range(8)]

    def weights(px, py, pz, gx, gy, gz):
        fx = px - gx.astype(jnp.float32)
        fy = py - gy.astype(jnp.float32)
        fz = pz - gz.astype(jnp.float32)
        wx = (1.0 - fx, fx)
        wy = (1.0 - fy, fy)
        wz = (1.0 - fz, fz)
        wxy = (wx[0] * wy[0], wx[1] * wy[0], wx[0] * wy[1], wx[1] * wy[1])
        return [wxy[c & 3] * wz[c >> 2] for c in range(8)]

    def scatter_out(l, o, acc0, acc1):
        prow = iota + o
        plsc.store_scatter(featv, [prow, jnp.full((16,), 2 * l, jnp.int32)], acc0)
        plsc.store_scatter(featv, [prow, jnp.full((16,), 2 * l + 1, jnp.int32)], acc1)

    def idx_phase(l, b):
        idxv = idxvs[b]
        colv = colvs[b]
        off = _OFF[l]

        def jb(j, carry):
            o = pl.multiple_of(j * 16, 16)
            _, _, _, gx, gy, gz = point_vregs(l, o)
            idxs = corner_indices(l, gx, gy, gz, off)
            for c in range(8):
                idxv[pl.ds(c * _C + o, 16)] = (idxs[c] >> 2) + tbase4
                colv[pl.ds(c * _C + o, 16)] = (idxs[c] & 3) * 2
            return carry

        lax.fori_loop(0, _C // 16, jb, 0, unroll=2)

    def fire(b):
        return pltpu.async_copy(tabi.at[idxvs[b]], rows[b], sems[b])

    def acc_phase(l, b):
        rv = rows[b]
        colv = colvs[b]

        def jb(j, carry):
            o = pl.multiple_of(j * 16, 16)
            px, py, pz, gx, gy, gz = point_vregs(l, o)
            ws = weights(px, py, pz, gx, gy, gz)
            acc0 = None
            acc1 = None
            for c in range(8):
                ridx = iota + (c * _C + o)
                cb = colv[pl.ds(c * _C + o, 16)]
                r0 = plsc.load_gather(rv, [ridx, cb])
                r1 = plsc.load_gather(rv, [ridx, cb + 1])
                if c == 0:
                    acc0 = ws[c] * r0
                    acc1 = ws[c] * r1
                else:
                    acc0 = acc0 + ws[c] * r0
                    acc1 = acc1 + ws[c] * r1
            scatter_out(l, o, acc0, acc1)
            return carry

        lax.fori_loop(0, _C // 16, jb, 0, unroll=2)

    def acc_local(l):
        off = _OFF[l]

        def jb(j, carry):
            o = pl.multiple_of(j * 16, 16)
            px, py, pz, gx, gy, gz = point_vregs(l, o)
            idxs = corner_indices(l, gx, gy, gz, off)
            ws = weights(px, py, pz, gx, gy, gz)
            acc0 = None
            acc1 = None
            for c in range(8):
                r0 = plsc.load_gather(t01a, [idxs[c]])
                r1 = plsc.load_gather(t01b, [idxs[c]])
                if c == 0:
                    acc0 = ws[c] * r0
                    acc1 = ws[c] * r1
                else:
                    acc0 = acc0 + ws[c] * r0
                    acc1 = acc1 + ws[c] * r1
            scatter_out(l, o, acc0, acc1)
            return carry

        lax.fori_loop(0, _C // 16, jb, 0, unroll=2)

    def chunk_body(ci, carry):
        base = pl.multiple_of(wid * _NPW + ci * _C, _C)
        pltpu.sync_copy(x0h.at[pl.ds(base, _C)], xv0)
        pltpu.sync_copy(x1h.at[pl.ds(base, _C)], xv1)
        pltpu.sync_copy(x2h.at[pl.ds(base, _C)], xv2)

        idx_phase(2, 0)
        handles = [None, None]
        handles[0] = fire(0)
        acc_local(0)
        acc_local(1)
        for l in range(3, _NUM_LEVELS):
            b = (l - 2) % 2
            bp = (l - 3) % 2
            idx_phase(l, b)
            handles[b] = fire(b)
            handles[bp].wait()
            acc_phase(l - 1, bp)
        bl = (_NUM_LEVELS - 3) % 2
        handles[bl].wait()
        acc_phase(_NUM_LEVELS - 1, bl)

        pltpu.sync_copy(featv, feath.at[pl.ds(base, _C)])
        return carry

    lax.fori_loop(0, _NCHUNK, chunk_body, 0)


def _build_body(t0h, t1h, tabi, tv0, tv1, ibuf):
    cid = lax.axis_index("c")
    sid = lax.axis_index("s")
    wid = sid * _NC + cid
    iota = lax.iota(jnp.int32, 16)

    def istep(k, carry):
        start = pl.multiple_of(wid * _IPW + k * _IC, _IC)
        pltpu.sync_copy(t0h.at[pl.ds(start, _IC)], tv0)
        pltpu.sync_copy(t1h.at[pl.ds(start, _IC)], tv1)

        def jb(j, c2):
            o = pl.multiple_of(j * 16, 16)
            rr = iota + o
            rq = rr >> 2
            rk = (rr & 3) * 2
            plsc.store_scatter(ibuf, [rq, rk], tv0[pl.ds(o, 16)])
            plsc.store_scatter(ibuf, [rq, rk + 1], tv1[pl.ds(o, 16)])
            return c2

        lax.fori_loop(0, _IC // 16, jb, 0, unroll=4)
        pltpu.sync_copy(
            ibuf, tabi.at[pl.ds((wid * _IPW + k * _IC) // 4, _IC // 4)])
        return carry

    lax.fori_loop(0, _ISTEPS, istep, 0)


@functools.partial(
    pl.kernel,
    out_type=jax.ShapeDtypeStruct((_TP // 4, 8), jnp.float32),
    mesh=plsc.VectorSubcoreMesh(core_axis_name="c", subcore_axis_name="s"),
    scratch_types=[
        pltpu.VMEM((_IC,), jnp.float32),
        pltpu.VMEM((_IC,), jnp.float32),
        pltpu.VMEM((_IC // 4, 8), jnp.float32),
    ],
    compiler_params=pltpu.CompilerParams(
        needs_layout_passes=False, use_tc_tiling_on_sc=False),
)
def _build_tab(*args):
    _build_body(*args)


@functools.partial(
    pl.kernel,
    out_type=jax.ShapeDtypeStruct((_NH, 32), jnp.float32),
    mesh=plsc.VectorSubcoreMesh(core_axis_name="c", subcore_axis_name="s"),
    scratch_types=[
        pltpu.VMEM((_C,), jnp.float32),
        pltpu.VMEM((_C,), jnp.float32),
        pltpu.VMEM((_C,), jnp.float32),
        pltpu.VMEM((8 * _C,), jnp.int32),
        pltpu.VMEM((8 * _C,), jnp.int32),
        pltpu.VMEM((8 * _C,), jnp.int32),
        pltpu.VMEM((8 * _C,), jnp.int32),
        pltpu.VMEM((8 * _C, 8), jnp.float32),
        pltpu.VMEM((8 * _C, 8), jnp.float32),
        pltpu.VMEM((_C, 32), jnp.float32),
        pltpu.VMEM((_NLOC,), jnp.float32),
        pltpu.VMEM((_NLOC,), jnp.float32),
        pltpu.SemaphoreType.DMA,
        pltpu.SemaphoreType.DMA,
    ],
    compiler_params=pltpu.CompilerParams(
        needs_layout_passes=False, use_tc_tiling_on_sc=False),
)
def _encode(*args):
    _enc_body(*args)


_B = 4096


def _mlp_body(fref, w0r, w1r, w2r, w3r, w4r, b0r, b1r, b2r, b3r, b4r, oref):
    dn = (((1,), (1,)), ((), ()))
    h = fref[...]
    h = jnp.maximum(
        lax.dot_general(h, w0r[...], dn, preferred_element_type=jnp.float32)
        + b0r[...], 0.0)
    h = jnp.maximum(
        lax.dot_general(h, w1r[...], dn, preferred_element_type=jnp.float32)
        + b1r[...], 0.0)
    h = jnp.maximum(
        lax.dot_general(h, w2r[...], dn, preferred_element_type=jnp.float32)
        + b2r[...], 0.0)
    h = jnp.maximum(
        lax.dot_general(h, w3r[...], dn, preferred_element_type=jnp.float32)
        + b3r[...], 0.0)
    out8 = lax.dot_general(h, w4r[...], dn, preferred_element_type=jnp.float32)
    oref[...] = out8[:, 0:1] + b4r[0, 0]


def _full_spec(shape):
    nd = len(shape)
    return pl.BlockSpec(shape, lambda i: (0,) * nd)


def _mlp(feats, W0, W1, W2, W3, W4, b0, b1, b2, b3, b4):
    grid = (_NH // _B,)
    return pl.pallas_call(
        _mlp_body,
        grid=grid,
        in_specs=[
            pl.BlockSpec((_B, 32), lambda i: (i, 0)),
            _full_spec(W0.shape), _full_spec(W1.shape), _full_spec(W2.shape),
            _full_spec(W3.shape), _full_spec(W4.shape),
            _full_spec(b0.shape), _full_spec(b1.shape), _full_spec(b2.shape),
            _full_spec(b3.shape),
            pl.BlockSpec(memory_space=pltpu.SMEM),
        ],
        out_specs=pl.BlockSpec((_B, 1), lambda i: (i, 0)),
        out_shape=jax.ShapeDtypeStruct((_NH, 1), jnp.float32),
    )(feats, W0, W1, W2, W3, W4, b0, b1, b2, b3, b4)


def kernel(x, table, W0, b0, W1, b1, W2, b2, W3, b3, W4, b4):
    x0 = x[:, 0]
    x1 = x[:, 1]
    x2 = x[:, 2]
    pad = _TP - _T
    t0 = jnp.pad(table[:, 0], (0, pad))
    t1 = jnp.pad(table[:, 1], (0, pad))
    tabi = _build_tab(t0, t1)
    W4p = jnp.pad(W4, ((0, 7), (0, 0)))
    bs = (b0.reshape(1, -1), b1.reshape(1, -1), b2.reshape(1, -1),
          b3.reshape(1, -1), b4.reshape(1, 1))
    outs = []
    for h in range(_NSPLIT):
        sl = slice(h * _NH, (h + 1) * _NH)
        feats = _encode(x0[sl], x1[sl], x2[sl], t0, t1, tabi)
        outs.append(_mlp(feats, W0, W1, W2, W3, W4p, *bs))
    return jnp.concatenate(outs, axis=0)


# speedup vs baseline: 2.3529x; 1.0079x over previous
import functools

import numpy as np
import jax
import jax.numpy as jnp
from jax import lax
from jax.experimental import pallas as pl
from jax.experimental.pallas import tpu as pltpu
from jax.experimental.pallas import tpu_sc as plsc

_NUM_LEVELS = 16
_BASE_RES = 16
_MAX_PARAMS = 2 ** 19
_DESIRED_RES = 2048
_N = 1048576
_PER_LEVEL_SCALE = float(np.exp2(np.log2(_DESIRED_RES / _BASE_RES) / (_NUM_LEVELS - 1)))
_P1 = int(np.uint32(2654435761).astype(np.int32))
_P2 = int(np.uint32(805459861).astype(np.int32))


def _levels():
    scales, resolutions, offsets, sizes = [], [], [], []
    offset = 0
    for l in range(_NUM_LEVELS):
        scale = _BASE_RES * (_PER_LEVEL_SCALE ** l) - 1.0
        res = int(np.ceil(scale)) + 1
        params = min(_MAX_PARAMS, res ** 3)
        params = int(np.ceil(params / 8) * 8)
        scales.append(scale)
        resolutions.append(res)
        offsets.append(offset)
        sizes.append(params)
        offset += params
    return scales, resolutions, offsets, sizes


_SCALES, _RES, _OFF, _SIZES = _levels()
_USE_HASH = [(r ** 3) > s for r, s in zip(_RES, _SIZES)]
_T = _OFF[-1] + _SIZES[-1]

_NC, _NS = 2, 16
_NW = _NC * _NS
_C = 256
_NSPLIT = 8
_NH = _N // _NSPLIT
_NPW = _NH // _NW
_NCHUNK = _NPW // _C

_IC = 512
_TP = ((_T + _NW * _IC - 1) // (_NW * _IC)) * (_NW * _IC)
_IPW = _TP // _NW
_ISTEPS = _IPW // _IC
_NLOC = _OFF[2]


def _enc_body(x0h, x1h, x2h, t0h, t1h, tabi, feath,
              xv0, xv1, xv2, idxv0, idxv1, colv0, colv1, ra, rb, featv,
              t01a, t01b, semA, semB):
    cid = lax.axis_index("c")
    sid = lax.axis_index("s")
    wid = sid * _NC + cid
    idxvs = (idxv0, idxv1)
    colvs = (colv0, colv1)
    rows = (ra, rb)
    sems = (semA, semB)
    iota = lax.iota(jnp.int32, 16)
    col0 = jnp.zeros((16,), jnp.int32)
    col1 = jnp.ones((16,), jnp.int32)
    tbase4 = 0

    pltpu.sync_copy(t0h.at[pl.ds(0, _NLOC)], t01a)
    pltpu.sync_copy(t1h.at[pl.ds(0, _NLOC)], t01b)

    def point_vregs(l, o):
        scale = jnp.float32(_SCALES[l])
        px = xv0[pl.ds(o, 16)] * scale + 0.5
        py = xv1[pl.ds(o, 16)] * scale + 0.5
        pz = xv2[pl.ds(o, 16)] * scale + 0.5
        gx = px.astype(jnp.int32)
        gy = py.astype(jnp.int32)
        gz = pz.astype(jnp.int32)
        return px, py, pz, gx, gy, gz

    def corner_indices(l, gx, gy, gz, off):
        gx1 = gx + 1
        gy1 = gy + 1
        gz1 = gz + 1
        if _USE_HASH[l]:
            m = _SIZES[l] - 1
            hy0 = gy * _P1
            hy1 = gy1 * _P1
            hz0 = gz * _P2
            hz1 = gz1 * _P2
            xy = (gx ^ hy0, gx1 ^ hy0, gx ^ hy1, gx1 ^ hy1)
            hz = (hz0, hz1)
            return [((xy[c & 3] ^ hz[c >> 2]) & m) + off for c in range(8)]
        res = _RES[l]
        sy0 = gy * res
        sy1 = gy1 * res
        sz0 = gz * (res * res)
        sz1 = gz1 * (res * res)
        yz = (sy0 + sz0, sy1 + sz0, sy0 + sz1, sy1 + sz1)
        gxs = (gx, gx1)
        return [gxs[c & 1] + yz[c >> 1] + off for c in range(8)]

    def weights(px, py, pz, gx, gy, gz):
        fx = px - gx.astype(jnp.float32)
        fy = py - gy.astype(jnp.float32)
        fz = pz - gz.astype(jnp.float32)
        wx = (1.0 - fx, fx)
        wy = (1.0 - fy, fy)
        wz = (1.0 - fz, fz)
        wxy = (wx[0] * wy[0], wx[1] * wy[0], wx[0] * wy[1], wx[1] * wy[1])
        return [wxy[c & 3] * wz[c >> 2] for c in range(8)]

    def scatter_out(l, o, acc0, acc1):
        prow = iota + o
        plsc.store_scatter(featv, [prow, jnp.full((16,), 2 * l, jnp.int32)], acc0)
        plsc.store_scatter(featv, [prow, jnp.full((16,), 2 * l + 1, jnp.int32)], acc1)

    def idx_phase(l, b):
        idxv = idxvs[b]
        colv = colvs[b]
        off = _OFF[l]

        def jb(j, carry):
            o = pl.multiple_of(j * 16, 16)
            _, _, _, gx, gy, gz = point_vregs(l, o)
            idxs = corner_indices(l, gx, gy, gz, off)
            for c in range(8):
                idxv[pl.ds(c * _C + o, 16)] = (idxs[c] >> 2) + tbase4
                colv[pl.ds(c * _C + o, 16)] = (idxs[c] & 3) * 2
            return carry

        lax.fori_loop(0, _C // 16, jb, 0, unroll=2)

    def fire(b):
        return pltpu.async_copy(tabi.at[idxvs[b]], rows[b], sems[b])

    def acc_phase(l, b):
        rv = rows[b]
        colv = colvs[b]

        def jb(j, carry):
            o = pl.multiple_of(j * 16, 16)
            px, py, pz, gx, gy, gz = point_vregs(l, o)
            ws = weights(px, py, pz, gx, gy, gz)
            acc0 = None
            acc1 = None
            for c in range(8):
                ridx = iota + (c * _C + o)
                cb = colv[pl.ds(c * _C + o, 16)]
                r0 = plsc.load_gather(rv, [ridx, cb])
                r1 = plsc.load_gather(rv, [ridx, cb + 1])
                if c == 0:
                    acc0 = ws[c] * r0
                    acc1 = ws[c] * r1
                else:
                    acc0 = acc0 + ws[c] * r0
                    acc1 = acc1 + ws[c] * r1
            scatter_out(l, o, acc0, acc1)
            return carry

        lax.fori_loop(0, _C // 16, jb, 0, unroll=2)

    def acc_local(l):
        off = _OFF[l]

        def jb(j, carry):
            o = pl.multiple_of(j * 16, 16)
            px, py, pz, gx, gy, gz = point_vregs(l, o)
            idxs = corner_indices(l, gx, gy, gz, off)
            ws = weights(px, py, pz, gx, gy, gz)
            acc0 = None
            acc1 = None
            for c in range(8):
                r0 = plsc.load_gather(t01a, [idxs[c]])
                r1 = plsc.load_gather(t01b, [idxs[c]])
                if c == 0:
                    acc0 = ws[c] * r0
                    acc1 = ws[c] * r1
                else:
                    acc0 = acc0 + ws[c] * r0
                    acc1 = acc1 + ws[c] * r1
            scatter_out(l, o, acc0, acc1)
            return carry

        lax.fori_loop(0, _C // 16, jb, 0, unroll=2)

    def chunk_body(ci, carry):
        base = pl.multiple_of(wid * _NPW + ci * _C, _C)
        pltpu.sync_copy(x0h.at[pl.ds(base, _C)], xv0)
        pltpu.sync_copy(x1h.at[pl.ds(base, _C)], xv1)
        pltpu.sync_copy(x2h.at[pl.ds(base, _C)], xv2)

        idx_phase(2, 0)
        handles = [None, None]
        handles[0] = fire(0)
        acc_local(0)
        acc_local(1)
        for l in range(3, _NUM_LEVELS):
            b = (l - 2) % 2
            bp = (l - 3) % 2
            idx_phase(l, b)
            handles[b] = fire(b)
            handles[bp].wait()
            acc_phase(l - 1, bp)
        bl = (_NUM_LEVELS - 3) % 2
        handles[bl].wait()
        acc_phase(_NUM_LEVELS - 1, bl)

        pltpu.sync_copy(featv, feath.at[pl.ds(base, _C)])
        return carry

    lax.fori_loop(0, _NCHUNK, chunk_body, 0)


def _build_body(t0h, t1h, tabi, tv0, tv1, ibuf):
    cid = lax.axis_index("c")
    sid = lax.axis_index("s")
    wid = sid * _NC + cid
    iota = lax.iota(jnp.int32, 16)

    def istep(k, carry):
        start = pl.multiple_of(wid * _IPW + k * _IC, _IC)
        pltpu.sync_copy(t0h.at[pl.ds(start, _IC)], tv0)
        pltpu.sync_copy(t1h.at[pl.ds(start, _IC)], tv1)

        def jb(j, c2):
            o = pl.multiple_of(j * 16, 16)
            rr = iota + o
            rq = rr >> 2
            rk = (rr & 3) * 2
            plsc.store_scatter(ibuf, [rq, rk], tv0[pl.ds(o, 16)])
            plsc.store_scatter(ibuf, [rq, rk + 1], tv1[pl.ds(o, 16)])
            return c2

        lax.fori_loop(0, _IC // 16, jb, 0, unroll=4)
        pltpu.sync_copy(
            ibuf, tabi.at[pl.ds((wid * _IPW + k * _IC) // 4, _IC // 4)])
        return carry

    lax.fori_loop(0, _ISTEPS, istep, 0)


@functools.partial(
    pl.kernel,
    out_type=jax.ShapeDtypeStruct((_TP // 4, 8), jnp.float32),
    mesh=plsc.VectorSubcoreMesh(core_axis_name="c", subcore_axis_name="s"),
    scratch_types=[
        pltpu.VMEM((_IC,), jnp.float32),
        pltpu.VMEM((_IC,), jnp.float32),
        pltpu.VMEM((_IC // 4, 8), jnp.float32),
    ],
    compiler_params=pltpu.CompilerParams(
        needs_layout_passes=False, use_tc_tiling_on_sc=False),
)
def _build_tab(*args):
    _build_body(*args)


@functools.partial(
    pl.kernel,
    out_type=jax.ShapeDtypeStruct((_NH, 32), jnp.float32),
    mesh=plsc.VectorSubcoreMesh(core_axis_name="c", subcore_axis_name="s"),
    scratch_types=[
        pltpu.VMEM((_C,), jnp.float32),
        pltpu.VMEM((_C,), jnp.float32),
        pltpu.VMEM((_C,), jnp.float32),
        pltpu.VMEM((8 * _C,), jnp.int32),
        pltpu.VMEM((8 * _C,), jnp.int32),
        pltpu.VMEM((8 * _C,), jnp.int32),
        pltpu.VMEM((8 * _C,), jnp.int32),
        pltpu.VMEM((8 * _C, 8), jnp.float32),
        pltpu.VMEM((8 * _C, 8), jnp.float32),
        pltpu.VMEM((_C, 32), jnp.float32),
        pltpu.VMEM((_NLOC,), jnp.float32),
        pltpu.VMEM((_NLOC,), jnp.float32),
        pltpu.SemaphoreType.DMA,
        pltpu.SemaphoreType.DMA,
    ],
    compiler_params=pltpu.CompilerParams(
        needs_layout_passes=False, use_tc_tiling_on_sc=False),
)
def _encode(*args):
    _enc_body(*args)


_B = 4096


def _mlp_body(fref, w0r, w1r, w2r, w3r, w4r, b0r, b1r, b2r, b3r, b4r, oref):
    dn = (((1,), (1,)), ((), ()))
    h = fref[...]
    h = jnp.maximum(
        lax.dot_general(h, w0r[...], dn, preferred_element_type=jnp.float32)
        + b0r[...], 0.0)
    h = jnp.maximum(
        lax.dot_general(h, w1r[...], dn, preferred_element_type=jnp.float32)
        + b1r[...], 0.0)
    h = jnp.maximum(
        lax.dot_general(h, w2r[...], dn, preferred_element_type=jnp.float32)
        + b2r[...], 0.0)
    h = jnp.maximum(
        lax.dot_general(h, w3r[...], dn, preferred_element_type=jnp.float32)
        + b3r[...], 0.0)
    out8 = lax.dot_general(h, w4r[...], dn, preferred_element_type=jnp.float32)
    oref[...] = out8[:, 0:1] + b4r[0, 0]


def _full_spec(shape):
    nd = len(shape)
    return pl.BlockSpec(shape, lambda i: (0,) * nd)


def _mlp(feats, W0, W1, W2, W3, W4, b0, b1, b2, b3, b4):
    grid = (_NH // _B,)
    return pl.pallas_call(
        _mlp_body,
        grid=grid,
        in_specs=[
            pl.BlockSpec((_B, 32), lambda i: (i, 0)),
            _full_spec(W0.shape), _full_spec(W1.shape), _full_spec(W2.shape),
            _full_spec(W3.shape), _full_spec(W4.shape),
            _full_spec(b0.shape), _full_spec(b1.shape), _full_spec(b2.shape),
            _full_spec(b3.shape),
            pl.BlockSpec(memory_space=pltpu.SMEM),
        ],
        out_specs=pl.BlockSpec((_B, 1), lambda i: (i, 0)),
        out_shape=jax.ShapeDtypeStruct((_NH, 1), jnp.float32),
    )(feats, W0, W1, W2, W3, W4, b0, b1, b2, b3, b4)


def kernel(x, table, W0, b0, W1, b1, W2, b2, W3, b3, W4, b4):
    x0 = x[:, 0]
    x1 = x[:, 1]
    x2 = x[:, 2]
    pad = _TP - _T
    t0 = jnp.pad(table[:, 0], (0, pad))
    t1 = jnp.pad(table[:, 1], (0, pad))
    tabi = _build_tab(t0, t1)
    W4p = jnp.pad(W4, ((0, 7), (0, 0)))
    bs = (b0.reshape(1, -1), b1.reshape(1, -1), b2.reshape(1, -1),
          b3.reshape(1, -1), b4.reshape(1, 1))
    outs = []
    for h in range(_NSPLIT):
        sl = slice(h * _NH, (h + 1) * _NH)
        feats = _encode(x0[sl], x1[sl], x2[sl], t0, t1, tabi)
        outs.append(_mlp(feats, W0, W1, W2, W3, W4p, *bs))
    return jnp.concatenate(outs, axis=0)
